# Initial kernel scaffold; baseline (speedup 1.0000x reference)
import jax
import jax.numpy as jnp
from jax.experimental import pallas as pl


def kernel(u, v, usr, ent, rel, W, b, adj_ent, adj_rel):
    raise NotImplementedError("write your pallas kernel here")


# speedup vs baseline: 13.9789x; 13.9789x over previous
import functools

import jax
import jax.numpy as jnp
from jax import lax
from jax.experimental import pallas as pl
from jax.experimental.pallas import tpu as pltpu
from jax.experimental.pallas import tpu_sc as plsc

NC = 2
NS = 16
NW = NC * NS
SB = 128

DIM = 64
K = 16


def _mesh():
    return plsc.VectorSubcoreMesh(core_axis_name="c", subcore_axis_name="s")


def _wid():
    return lax.axis_index("s") * NC + lax.axis_index("c")


def _make_row_gather(n_idx, ncols, dtype, n_tables, group):
    total_sb = n_idx // SB
    sb_per_w = total_sb // NW
    g = min(group, sb_per_w)
    ngrp = sb_per_w // g

    scratch = [pltpu.VMEM((g, SB), jnp.int32)]
    scratch += [pltpu.VMEM((g * SB, ncols), dtype) for _ in range(n_tables)]
    scratch += [pltpu.SemaphoreType.DMA]

    out_type = [jax.ShapeDtypeStruct((n_idx, ncols), dtype) for _ in range(n_tables)]
    if n_tables == 1:
        out_type = out_type[0]

    @functools.partial(
        pl.kernel,
        out_type=out_type,
        mesh=_mesh(),
        scratch_types=scratch,
        compiler_params=pltpu.CompilerParams(use_tc_tiling_on_sc=False),
    )
    def kern(idx_hbm, *refs):
        tables = refs[:n_tables]
        outs = refs[n_tables:2 * n_tables]
        idx_v = refs[2 * n_tables]
        rows_v = refs[2 * n_tables + 1:2 * n_tables + 1 + n_tables]
        sem = refs[-1]
        base_sb = _wid() * sb_per_w

        def body(i, carry):
            sb0 = base_sb + i * g
            pltpu.sync_copy(idx_hbm.at[pl.ds(sb0, g)], idx_v)
            copies = []
            for t in range(n_tables):
                for j in range(g):
                    copies.append(pltpu.async_copy(
                        tables[t].at[idx_v.at[j]],
                        rows_v[t].at[pl.ds(j * SB, SB)], sem))
            for c in copies:
                c.wait()
            for t in range(n_tables):
                pltpu.sync_copy(rows_v[t], outs[t].at[pl.ds(sb0 * SB, g * SB)])
            return carry

        lax.fori_loop(0, ngrp, body, 0, unroll=False)

    return kern


def _make_weighted_agg(n_out, chunk):
    r_per_w = n_out // NW
    nchunk = r_per_w // chunk
    sb_per_chunk = chunk * K // SB

    @functools.partial(
        pl.kernel,
        out_type=jax.ShapeDtypeStruct((n_out, DIM), jnp.float32),
        mesh=_mesh(),
        compiler_params=pltpu.CompilerParams(use_tc_tiling_on_sc=False),
        scratch_types=[
            pltpu.VMEM((sb_per_chunk, SB), jnp.int32),
            pltpu.VMEM((chunk * K,), jnp.float32),
            pltpu.VMEM((chunk * K, DIM), jnp.float32),
            pltpu.VMEM((chunk, DIM), jnp.float32),
            pltpu.SemaphoreType.DMA,
        ],
    )
    def kern(idx_hbm, w_hbm, table_hbm, out_hbm, idx_v, w_v, rows_v, acc_v, sem):
        base = _wid() * r_per_w

        def chunk_body(ci, carry):
            off = base + ci * chunk
            sb0 = off * K // SB
            pltpu.sync_copy(idx_hbm.at[pl.ds(sb0, sb_per_chunk)], idx_v)
            pltpu.sync_copy(w_hbm.at[pl.ds(off * K, chunk * K)], w_v)
            copies = []
            for j in range(sb_per_chunk):
                copies.append(pltpu.async_copy(
                    table_hbm.at[idx_v.at[j]],
                    rows_v.at[pl.ds(j * SB, SB)], sem))
            for c in copies:
                c.wait()

            def row_body(r, carry2):
                rbase = pl.multiple_of(r * K, K)
                wvec = w_v[pl.ds(rbase, 16)]
                accs = [jnp.zeros((16,), jnp.float32) for _ in range(DIM // 16)]
                dnums = lax.GatherDimensionNumbers(
                    offset_dims=(), collapsed_slice_dims=(0,),
                    start_index_map=(0,))
                for kk in range(K):
                    wb = lax.gather(
                        wvec, jnp.full((16, 1), kk, jnp.int32), dnums, (1,),
                        mode=lax.GatherScatterMode.PROMISE_IN_BOUNDS)
                    for d in range(DIM // 16):
                        accs[d] = accs[d] + wb * rows_v[rbase + kk, pl.ds(d * 16, 16)]
                for d in range(DIM // 16):
                    acc_v[r, pl.ds(d * 16, 16)] = accs[d]
                return carry2

            lax.fori_loop(0, chunk, row_body, 0, unroll=False)
            pltpu.sync_copy(acc_v, out_hbm.at[pl.ds(off, chunk)])
            return carry

        lax.fori_loop(0, nchunk, chunk_body, 0, unroll=False)

    return kern


NUM_REL_PAD = 32


def _tc_weights_body(ue_ref, rel_ref, nr1_ref, nr2_ref, w1_ref, w2_ref):
    ue = ue_ref[...]
    relm = rel_ref[...]
    urs = lax.dot_general(ue, relm, (((1,), (1,)), ((), ())),
                          preferred_element_type=jnp.float32)
    nr1 = nr1_ref[...]
    nr2 = nr2_ref[...]
    tb = ue.shape[0]
    s1 = jnp.zeros((tb, K), jnp.float32)
    s2 = jnp.zeros((tb, K * K), jnp.float32)
    nrel = relm.shape[0]
    for r in range(nrel):
        c = urs[:, r]
        s1 = s1 + jnp.where(nr1 == r, c[:, None], 0.0)
        s2 = s2 + jnp.where(nr2 == r, c[:, None], 0.0)

    e1 = jnp.exp(s1)
    w1_ref[...] = e1 / jnp.sum(e1, axis=-1, keepdims=True)

    e2 = jnp.exp(s2)
    seg = (lax.broadcasted_iota(jnp.int32, (K * K, K), 0) // K ==
           lax.broadcasted_iota(jnp.int32, (K * K, K), 1)).astype(jnp.float32)
    z = lax.dot_general(e2, seg, (((1,), (0,)), ((), ())),
                        preferred_element_type=jnp.float32)
    zb = lax.dot_general(z, seg, (((1,), (1,)), ((), ())),
                         preferred_element_type=jnp.float32)
    w2_ref[...] = e2 / zb


def _tc_final_body(e0_ref, e1_ref, agg2_ref, w1_ref, ue_ref, wt_ref, b_ref, out_ref):
    tb = e0_ref.shape[0]
    wt = wt_ref[...]
    bias = b_ref[...]
    e1 = e1_ref[...]
    x1 = e1 + agg2_ref[...]
    h1 = jax.nn.sigmoid(
        lax.dot_general(x1, wt, (((1,), (0,)), ((), ())),
                        preferred_element_type=jnp.float32) + bias)
    w1 = w1_ref[...]
    e1r = e1.reshape(tb, K, DIM)
    agg1 = jnp.sum(w1[..., None] * e1r, axis=1)
    h0 = jax.nn.sigmoid(
        lax.dot_general(e0_ref[...] + agg1, wt, (((1,), (0,)), ((), ())),
                        preferred_element_type=jnp.float32) + bias)
    h1r = h1.reshape(tb, K, DIM)
    aggf = jnp.sum(w1[..., None] * h1r, axis=1)
    fin = jnp.tanh(
        lax.dot_general(h0 + aggf, wt, (((1,), (0,)), ((), ())),
                        preferred_element_type=jnp.float32) + bias)
    out_ref[...] = jax.nn.sigmoid(jnp.sum(ue_ref[...] * fin, axis=-1))


def kernel(u, v, usr, ent, rel, W, b, adj_ent, adj_rel):
    B = u.shape[0]
    u = u.astype(jnp.int32)
    v = v.astype(jnp.int32)

    v_sb = v.reshape(B // SB, SB)
    ue = _make_row_gather(B, DIM, jnp.float32, 1, group=1)(u.reshape(B // SB, SB), usr)
    e0 = _make_row_gather(B, DIM, jnp.float32, 1, group=1)(v_sb, ent)
    ne1, nr1 = _make_row_gather(B, K, jnp.int32, 2, group=1)(v_sb, adj_ent, adj_rel)

    ne1f = ne1.reshape(B * K // SB, SB)
    ne2, nr2 = _make_row_gather(B * K, K, jnp.int32, 2, group=8)(ne1f, adj_ent, adj_rel)
    e1 = _make_row_gather(B * K, DIM, jnp.float32, 1, group=8)(ne1f, ent)

    TB = 256
    grid = (B // TB,)
    w1, w2 = pl.pallas_call(
        _tc_weights_body,
        grid=grid,
        in_specs=[
            pl.BlockSpec((TB, DIM), lambda i: (i, 0)),
            pl.BlockSpec((rel.shape[0], DIM), lambda i: (0, 0)),
            pl.BlockSpec((TB, K), lambda i: (i, 0)),
            pl.BlockSpec((TB, K * K), lambda i: (i, 0)),
        ],
        out_specs=[
            pl.BlockSpec((TB, K), lambda i: (i, 0)),
            pl.BlockSpec((TB, K * K), lambda i: (i, 0)),
        ],
        out_shape=[
            jax.ShapeDtypeStruct((B, K), jnp.float32),
            jax.ShapeDtypeStruct((B, K * K), jnp.float32),
        ],
    )(ue, rel, nr1, nr2.reshape(B, K * K))

    agg2 = _make_weighted_agg(B * K, chunk=32)(
        ne2.reshape(B * K * K // SB, SB), w2.reshape(B * K * K), ent)

    out = pl.pallas_call(
        _tc_final_body,
        grid=grid,
        in_specs=[
            pl.BlockSpec((TB, DIM), lambda i: (i, 0)),
            pl.BlockSpec((TB * K, DIM), lambda i: (i, 0)),
            pl.BlockSpec((TB * K, DIM), lambda i: (i, 0)),
            pl.BlockSpec((TB, K), lambda i: (i, 0)),
            pl.BlockSpec((TB, DIM), lambda i: (i, 0)),
            pl.BlockSpec((DIM, DIM), lambda i: (0, 0)),
            pl.BlockSpec((1, DIM), lambda i: (0, 0)),
        ],
        out_specs=pl.BlockSpec((TB,), lambda i: (i,)),
        out_shape=jax.ShapeDtypeStruct((B,), jnp.float32),
    )(e0, e1, agg2, w1, ue, W.T, b.reshape(1, DIM))
    return out


# speedup vs baseline: 17.0312x; 1.2184x over previous
import functools

import jax
import jax.numpy as jnp
from jax import lax
from jax.experimental import pallas as pl
from jax.experimental.pallas import tpu as pltpu
from jax.experimental.pallas import tpu_sc as plsc

NC = 2
NS = 16
NW = NC * NS
SB = 128

DIM = 64
K = 16

_SC_PARAMS = None


def _mesh():
    return plsc.VectorSubcoreMesh(core_axis_name="c", subcore_axis_name="s")


def _wid():
    return lax.axis_index("s") * NC + lax.axis_index("c")


def _sc_params():
    return pltpu.CompilerParams(use_tc_tiling_on_sc=False)


def _make_gather_hop0(B):
    assert B == NW * SB

    @functools.partial(
        pl.kernel,
        out_type=[
            jax.ShapeDtypeStruct((B, DIM), jnp.float32),
            jax.ShapeDtypeStruct((B, DIM), jnp.float32),
            jax.ShapeDtypeStruct((B, K), jnp.int32),
            jax.ShapeDtypeStruct((B, K), jnp.int32),
        ],
        mesh=_mesh(),
        compiler_params=_sc_params(),
        scratch_types=[
            pltpu.VMEM((1, SB), jnp.int32),
            pltpu.VMEM((1, SB), jnp.int32),
            pltpu.VMEM((SB, DIM), jnp.float32),
            pltpu.VMEM((SB, DIM), jnp.float32),
            pltpu.VMEM((SB, K), jnp.int32),
            pltpu.VMEM((SB, K), jnp.int32),
            pltpu.SemaphoreType.DMA,
        ],
    )
    def kern(u_hbm, v_hbm, usr_hbm, ent_hbm, ae_hbm, ar_hbm,
             ue_out, e0_out, ne1_out, nr1_out,
             ui_v, vi_v, ue_v, e0_v, ne1_v, nr1_v, sem):
        sb0 = _wid()
        pltpu.sync_copy(u_hbm.at[pl.ds(sb0, 1)], ui_v)
        pltpu.sync_copy(v_hbm.at[pl.ds(sb0, 1)], vi_v)
        cs = [
            pltpu.async_copy(usr_hbm.at[ui_v.at[0]], ue_v, sem),
            pltpu.async_copy(ent_hbm.at[vi_v.at[0]], e0_v, sem),
            pltpu.async_copy(ae_hbm.at[vi_v.at[0]], ne1_v, sem),
            pltpu.async_copy(ar_hbm.at[vi_v.at[0]], nr1_v, sem),
        ]
        for c in cs:
            c.wait()
        row0 = sb0 * SB
        pltpu.sync_copy(ue_v, ue_out.at[pl.ds(row0, SB)])
        pltpu.sync_copy(e0_v, e0_out.at[pl.ds(row0, SB)])
        pltpu.sync_copy(ne1_v, ne1_out.at[pl.ds(row0, SB)])
        pltpu.sync_copy(nr1_v, nr1_out.at[pl.ds(row0, SB)])

    return kern


def _make_gather_hop1(n_idx, group):
    total_sb = n_idx // SB
    sb_per_w = total_sb // NW
    g = min(group, sb_per_w)
    ngrp = sb_per_w // g

    @functools.partial(
        pl.kernel,
        out_type=[
            jax.ShapeDtypeStruct((n_idx, K), jnp.int32),
            jax.ShapeDtypeStruct((n_idx, K), jnp.int32),
            jax.ShapeDtypeStruct((n_idx, DIM), jnp.float32),
        ],
        mesh=_mesh(),
        compiler_params=_sc_params(),
        scratch_types=[
            pltpu.VMEM((g, SB), jnp.int32),
            pltpu.VMEM((g * SB, K), jnp.int32),
            pltpu.VMEM((g * SB, K), jnp.int32),
            pltpu.VMEM((g * SB, DIM), jnp.float32),
            pltpu.SemaphoreType.DMA,
        ],
    )
    def kern(idx_hbm, ae_hbm, ar_hbm, ent_hbm,
             ne2_out, nr2_out, e1_out,
             idx_v, ne2_v, nr2_v, e1_v, sem):
        base_sb = _wid() * sb_per_w

        def body(i, carry):
            sb0 = base_sb + i * g
            pltpu.sync_copy(idx_hbm.at[pl.ds(sb0, g)], idx_v)
            cs = []
            for j in range(g):
                cs.append(pltpu.async_copy(
                    ae_hbm.at[idx_v.at[j]], ne2_v.at[pl.ds(j * SB, SB)], sem))
                cs.append(pltpu.async_copy(
                    ar_hbm.at[idx_v.at[j]], nr2_v.at[pl.ds(j * SB, SB)], sem))
                cs.append(pltpu.async_copy(
                    ent_hbm.at[idx_v.at[j]], e1_v.at[pl.ds(j * SB, SB)], sem))
            for c in cs:
                c.wait()
            row0 = sb0 * SB
            pltpu.sync_copy(ne2_v, ne2_out.at[pl.ds(row0, g * SB)])
            pltpu.sync_copy(nr2_v, nr2_out.at[pl.ds(row0, g * SB)])
            pltpu.sync_copy(e1_v, e1_out.at[pl.ds(row0, g * SB)])
            return carry

        lax.fori_loop(0, ngrp, body, 0, unroll=False)

    return kern


_BC = 256
_SUBC = 32


def _make_weighted_agg(n_out):
    r_per_w = n_out // NW
    nbig = r_per_w // _BC
    nsub = _BC // _SUBC
    sb_per_sub = _SUBC * K // SB

    @functools.partial(
        pl.kernel,
        out_type=jax.ShapeDtypeStruct((n_out, DIM), jnp.float32),
        mesh=_mesh(),
        compiler_params=_sc_params(),
        scratch_types=[
            pltpu.VMEM((_BC * K // SB, SB), jnp.int32),
            pltpu.VMEM((_BC * K,), jnp.float32),
            pltpu.VMEM((_SUBC * K, DIM), jnp.float32),
            pltpu.VMEM((_SUBC * K, DIM), jnp.float32),
            pltpu.VMEM((_BC, DIM), jnp.float32),
            pltpu.SemaphoreType.DMA,
            pltpu.SemaphoreType.DMA,
        ],
    )
    def kern(idx_hbm, w_hbm, table_hbm, out_hbm,
             idx_v, w_v, rows0, rows1, acc_v, sem0, sem1):
        base = _wid() * r_per_w
        rows = (rows0, rows1)
        sems = (sem0, sem1)

        def fire(sc, buf):
            return [pltpu.async_copy(
                table_hbm.at[idx_v.at[sc * sb_per_sub + j]],
                rows[buf].at[pl.ds(j * SB, SB)], sems[buf])
                for j in range(sb_per_sub)]

        dnums = lax.GatherDimensionNumbers(
            offset_dims=(), collapsed_slice_dims=(0,), start_index_map=(0,))

        def compute(sc, rref):
            def row_body(r, carry2):
                roff = pl.multiple_of(r * K, K)
                woff = pl.multiple_of(sc * _SUBC * K, K) + roff
                wvec = w_v[pl.ds(woff, 16)]
                acc_a = [jnp.zeros((16,), jnp.float32) for _ in range(DIM // 16)]
                acc_b = [jnp.zeros((16,), jnp.float32) for _ in range(DIM // 16)]
                for kk in range(K):
                    wb = lax.gather(
                        wvec, jnp.full((16, 1), kk, jnp.int32), dnums, (1,),
                        mode=lax.GatherScatterMode.PROMISE_IN_BOUNDS)
                    tgt = acc_a if kk % 2 == 0 else acc_b
                    for d in range(DIM // 16):
                        tgt[d] = tgt[d] + wb * rref[roff + kk, pl.ds(d * 16, 16)]
                arow = sc * _SUBC + r
                for d in range(DIM // 16):
                    acc_v[arow, pl.ds(d * 16, 16)] = acc_a[d] + acc_b[d]
                return carry2

            lax.fori_loop(0, _SUBC, row_body, 0, unroll=2)

        def big_body(bi, carry):
            off = base + bi * _BC
            pltpu.sync_copy(idx_hbm.at[pl.ds(off * K // SB, _BC * K // SB)], idx_v)
            pltpu.sync_copy(w_hbm.at[pl.ds(off * K, _BC * K)], w_v)
            pending = {0: fire(0, 0)}
            for sc in range(nsub):
                buf = sc & 1
                if sc + 1 < nsub:
                    pending[1 - buf] = fire(sc + 1, 1 - buf)
                for c in pending[buf]:
                    c.wait()
                compute(sc, rows[buf])
            pltpu.sync_copy(acc_v, out_hbm.at[pl.ds(off, _BC)])
            return carry

        lax.fori_loop(0, nbig, big_body, 0, unroll=False)

    return kern


def _tc_weights_body(ue_ref, rel_ref, nr1_ref, nr2_ref, w1_ref, w2_ref):
    ue = ue_ref[...]
    relm = rel_ref[...]
    urs = lax.dot_general(ue, relm, (((1,), (1,)), ((), ())),
                          preferred_element_type=jnp.float32)
    nr1 = nr1_ref[...]
    nr2 = nr2_ref[...]
    tb = ue.shape[0]
    s1 = jnp.zeros((tb, K), jnp.float32)
    s2 = jnp.zeros((tb, K * K), jnp.float32)
    nrel = relm.shape[0]
    for r in range(nrel):
        c = urs[:, r]
        s1 = s1 + jnp.where(nr1 == r, c[:, None], 0.0)
        s2 = s2 + jnp.where(nr2 == r, c[:, None], 0.0)

    e1 = jnp.exp(s1)
    w1_ref[...] = e1 / jnp.sum(e1, axis=-1, keepdims=True)

    e2 = jnp.exp(s2)
    seg = (lax.broadcasted_iota(jnp.int32, (K * K, K), 0) // K ==
           lax.broadcasted_iota(jnp.int32, (K * K, K), 1)).astype(jnp.float32)
    z = lax.dot_general(e2, seg, (((1,), (0,)), ((), ())),
                        preferred_element_type=jnp.float32)
    zb = lax.dot_general(z, seg, (((1,), (1,)), ((), ())),
                         preferred_element_type=jnp.float32)
    w2_ref[...] = e2 / zb


def _tc_final_body(e0_ref, e1_ref, agg2_ref, w1_ref, ue_ref, wt_ref, b_ref, out_ref):
    tb = e0_ref.shape[0]
    wt = wt_ref[...]
    bias = b_ref[...]
    e1 = e1_ref[...]
    x1 = e1 + agg2_ref[...]
    h1 = jax.nn.sigmoid(
        lax.dot_general(x1, wt, (((1,), (0,)), ((), ())),
                        preferred_element_type=jnp.float32) + bias)
    w1 = w1_ref[...]
    e1r = e1.reshape(tb, K, DIM)
    agg1 = jnp.sum(w1[..., None] * e1r, axis=1)
    h0 = jax.nn.sigmoid(
        lax.dot_general(e0_ref[...] + agg1, wt, (((1,), (0,)), ((), ())),
                        preferred_element_type=jnp.float32) + bias)
    h1r = h1.reshape(tb, K, DIM)
    aggf = jnp.sum(w1[..., None] * h1r, axis=1)
    fin = jnp.tanh(
        lax.dot_general(h0 + aggf, wt, (((1,), (0,)), ((), ())),
                        preferred_element_type=jnp.float32) + bias)
    out_ref[...] = jax.nn.sigmoid(jnp.sum(ue_ref[...] * fin, axis=-1))


def kernel(u, v, usr, ent, rel, W, b, adj_ent, adj_rel):
    B = u.shape[0]
    u = u.astype(jnp.int32)
    v = v.astype(jnp.int32)

    ue, e0, ne1, nr1 = _make_gather_hop0(B)(
        u.reshape(B // SB, SB), v.reshape(B // SB, SB), usr, ent,
        adj_ent, adj_rel)

    ne1f = ne1.reshape(B * K // SB, SB)
    ne2, nr2, e1 = _make_gather_hop1(B * K, group=8)(
        ne1f, adj_ent, adj_rel, ent)

    TB = 256
    grid = (B // TB,)
    w1, w2 = pl.pallas_call(
        _tc_weights_body,
        grid=grid,
        in_specs=[
            pl.BlockSpec((TB, DIM), lambda i: (i, 0)),
            pl.BlockSpec((rel.shape[0], DIM), lambda i: (0, 0)),
            pl.BlockSpec((TB, K), lambda i: (i, 0)),
            pl.BlockSpec((TB, K * K), lambda i: (i, 0)),
        ],
        out_specs=[
            pl.BlockSpec((TB, K), lambda i: (i, 0)),
            pl.BlockSpec((TB, K * K), lambda i: (i, 0)),
        ],
        out_shape=[
            jax.ShapeDtypeStruct((B, K), jnp.float32),
            jax.ShapeDtypeStruct((B, K * K), jnp.float32),
        ],
    )(ue, rel, nr1, nr2.reshape(B, K * K))

    agg2 = _make_weighted_agg(B * K)(
        ne2.reshape(B * K * K // SB, SB), w2.reshape(B * K * K), ent)

    out = pl.pallas_call(
        _tc_final_body,
        grid=grid,
        in_specs=[
            pl.BlockSpec((TB, DIM), lambda i: (i, 0)),
            pl.BlockSpec((TB * K, DIM), lambda i: (i, 0)),
            pl.BlockSpec((TB * K, DIM), lambda i: (i, 0)),
            pl.BlockSpec((TB, K), lambda i: (i, 0)),
            pl.BlockSpec((TB, DIM), lambda i: (i, 0)),
            pl.BlockSpec((DIM, DIM), lambda i: (0, 0)),
            pl.BlockSpec((1, DIM), lambda i: (0, 0)),
        ],
        out_specs=pl.BlockSpec((TB,), lambda i: (i,)),
        out_shape=jax.ShapeDtypeStruct((B,), jnp.float32),
    )(e0, e1, agg2, w1, ue, W.T, b.reshape(1, DIM))
    return out
